# baseline (speedup 1.0000x reference)
import functools

import jax
import jax.numpy as jnp
from jax import lax
from jax.experimental import pallas as pl
from jax.experimental.pallas import tpu as pltpu
from jax.experimental.pallas import tpu_sc as plsc

_B, _F, _V, _D = 1024, 26, 100000, 16
_IN = _F * _D
_NC, _NS, _L = 2, 16, 16
_NW = _NC * _NS
_HB = _B // 2
_BPW = _HB // _NW
_CHUNK = _BPW * _F


_NG = _CHUNK // _L


def _sc_gather(tab_hbm, x_hbm, out_hbm, xv, slabs, rows_v, sem):
    wid = lax.axis_index("s") * _NC + lax.axis_index("c")
    pltpu.sync_copy(x_hbm.at[pl.ds(wid * _CHUNK, _CHUNK)], xv)
    dlane = lax.iota(jnp.int32, _L)

    def issue(j, buf):
        vec = xv[pl.ds(j * _L, _L)]
        for i in range(_L):
            l = j * _L + i
            f = lax.rem(l, _F)
            tc = lax.div(vec[i], 128)
            pltpu.async_copy(
                tab_hbm.at[pl.ds(pl.multiple_of(f * _D, 8), _D),
                           pl.ds(pl.multiple_of(tc * 128, 128), 128)],
                slabs.at[buf, pl.ds(i * _D, _D)], sem)

    def extract(j, buf):
        vec = xv[pl.ds(j * _L, _L)]
        bufv = jnp.full((_L,), buf, jnp.int32)
        for i in range(_L):
            l = j * _L + i
            f = lax.rem(l, _F)
            br = lax.div(l, _F)
            vh = lax.rem(vec[i], 128)
            col = plsc.load_gather(
                slabs, [bufv, i * _D + dlane, jnp.full((_L,), vh, jnp.int32)])
            plsc.store_scatter(
                rows_v, [jnp.full((_L,), br, jnp.int32), f * _D + dlane], col)

    def drain(buf):
        pltpu.make_async_copy(
            tab_hbm.at[pl.ds(0, _L * _D), pl.ds(0, 128)],
            slabs.at[buf], sem).wait()

    issue(0, 0)

    def body(j, carry):
        buf = lax.rem(j, 2)
        issue(j, buf)
        drain(1 - buf)
        extract(j - 1, 1 - buf)
        return carry

    lax.fori_loop(1, _NG, body, 0, unroll=False)
    last = lax.rem(jnp.int32(_NG - 1), 2)
    drain(last)
    extract(_NG - 1, last)
    pltpu.sync_copy(rows_v, out_hbm.at[pl.ds(wid * _BPW, _BPW)])


@jax.jit
def _gather(tab_t, x_flat):
    mesh = plsc.VectorSubcoreMesh(
        core_axis_name="c", subcore_axis_name="s", num_cores=_NC, num_subcores=_NS
    )
    return pl.kernel(
        _sc_gather,
        out_type=jax.ShapeDtypeStruct((_HB, _IN), jnp.float32),
        name="emb_slab_gather",
        mesh=mesh,
        compiler_params=pltpu.CompilerParams(
            needs_layout_passes=False, disable_bounds_checks=True),
        scratch_types=[
            pltpu.VMEM((_CHUNK,), jnp.int32),
            pltpu.VMEM((2, _L * _D, 128), jnp.float32),
            pltpu.VMEM((_BPW, _IN), jnp.float32),
            pltpu.SemaphoreType.DMA,
        ],
    )(tab_t, x_flat)


def _tc_body(e_ref, a_ref, w1_ref, b1_ref, w2_ref, b2_ref, w3_ref, b3_ref,
             w4_ref, b4_ref, out_ref):
    e = e_ref[...]
    t = jnp.dot(e, a_ref[...], preferred_element_type=jnp.float32)
    fwfm = 0.5 * jnp.sum(e * t, axis=1, keepdims=True)
    h = jnp.maximum(
        jnp.dot(e, w1_ref[...], preferred_element_type=jnp.float32) + b1_ref[...], 0.0)
    h = jnp.maximum(
        jnp.dot(h, w2_ref[...], preferred_element_type=jnp.float32) + b2_ref[...], 0.0)
    h = jnp.maximum(
        jnp.dot(h, w3_ref[...], preferred_element_type=jnp.float32) + b3_ref[...], 0.0)
    o = jnp.dot(h, w4_ref[...], preferred_element_type=jnp.float32) + b4_ref[...]
    out_ref[...] = jax.nn.sigmoid(fwfm + o)


@jax.jit
def _tc_head(emb, a_big, W1, b1, W2, b2, W3, b3, W4, b4):
    return pl.pallas_call(
        _tc_body,
        out_shape=jax.ShapeDtypeStruct((_HB, 1), jnp.float32),
    )(emb, a_big, W1, b1, W2, b2, W3, b3, W4, b4)


def kernel(x, tables, field_cov, W1, b1, W2, b2, W3, b3, W4, b4, bias):
    tab_t = tables.transpose(0, 2, 1).reshape(_IN, _V)
    x_flat = x.reshape(-1)
    emb0 = _gather(tab_t, x_flat[: _HB * _F])
    emb1 = _gather(tab_t, x_flat[_HB * _F:])

    sym = (field_cov.T + field_cov) * 0.5 - jnp.eye(_F, dtype=jnp.float32)
    a_big = jnp.kron(sym, jnp.eye(_D, dtype=jnp.float32))
    ws = (W1, b1.reshape(1, -1), W2, b2.reshape(1, -1),
          W3, b3.reshape(1, -1), W4, (b4 + bias).reshape(1, 1))
    out0 = _tc_head(emb0, a_big, *ws)
    out1 = _tc_head(emb1, a_big, *ws)
    return jnp.concatenate([out0[:, 0], out1[:, 0]])


# speedup vs baseline: 1.0616x; 1.0616x over previous
import jax
import jax.numpy as jnp
from jax import lax
from jax.experimental import pallas as pl
from jax.experimental.pallas import tpu as pltpu
from jax.experimental.pallas import tpu_sc as plsc

_B, _F, _V, _D = 1024, 26, 100000, 16
_IN = _F * _D
_NC, _NS, _L = 2, 16, 16
_NW = _NC * _NS
_BPW = _B // _NW
_CHUNK = _BPW * _F


_NG = _CHUNK // _L


def _sc_gather(tab_hbm, x_hbm, out_hbm, xv, slabs, rows_v, sem):
    wid = lax.axis_index("s") * _NC + lax.axis_index("c")
    pltpu.sync_copy(x_hbm.at[pl.ds(wid * _CHUNK, _CHUNK)], xv)
    dlane = lax.iota(jnp.int32, _L)

    def issue(j, buf):
        vec = xv[pl.ds(j * _L, _L)]
        for i in range(_L):
            l = j * _L + i
            f = lax.rem(l, _F)
            tc = lax.div(vec[i], 128)
            pltpu.async_copy(
                tab_hbm.at[pl.ds(pl.multiple_of(f * _D, 8), _D),
                           pl.ds(pl.multiple_of(tc * 128, 128), 128)],
                slabs.at[buf, pl.ds(i * _D, _D)], sem)

    def extract(j, buf):
        vec = xv[pl.ds(j * _L, _L)]
        bufv = jnp.full((_L,), buf, jnp.int32)
        for i in range(_L):
            l = j * _L + i
            f = lax.rem(l, _F)
            br = lax.div(l, _F)
            vh = lax.rem(vec[i], 128)
            col = plsc.load_gather(
                slabs, [bufv, i * _D + dlane, jnp.full((_L,), vh, jnp.int32)])
            plsc.store_scatter(
                rows_v, [jnp.full((_L,), br, jnp.int32), f * _D + dlane], col)

    def drain(buf):
        pltpu.make_async_copy(
            tab_hbm.at[pl.ds(0, _L * _D), pl.ds(0, 128)],
            slabs.at[buf], sem).wait()

    issue(0, 0)

    def body(j, carry):
        buf = lax.rem(j, 2)
        issue(j, buf)
        drain(1 - buf)
        extract(j - 1, 1 - buf)
        return carry

    lax.fori_loop(1, _NG, body, 0, unroll=False)
    last = lax.rem(jnp.int32(_NG - 1), 2)
    drain(last)
    extract(_NG - 1, last)
    pltpu.sync_copy(rows_v, out_hbm.at[pl.ds(wid * _BPW, _BPW)])


@jax.jit
def _gather(tab_t, x_flat):
    mesh = plsc.VectorSubcoreMesh(
        core_axis_name="c", subcore_axis_name="s", num_cores=_NC, num_subcores=_NS
    )
    return pl.kernel(
        _sc_gather,
        out_type=jax.ShapeDtypeStruct((_B, _IN), jnp.float32),
        name="emb_slab_gather",
        mesh=mesh,
        compiler_params=pltpu.CompilerParams(
            needs_layout_passes=False, disable_bounds_checks=True),
        scratch_types=[
            pltpu.VMEM((_CHUNK,), jnp.int32),
            pltpu.VMEM((2, _L * _D, 128), jnp.float32),
            pltpu.VMEM((_BPW, _IN), jnp.float32),
            pltpu.SemaphoreType.DMA,
        ],
    )(tab_t, x_flat)


def _tc_body(e_ref, a_ref, w1_ref, b1_ref, w2_ref, b2_ref, w3_ref, b3_ref,
             w4_ref, b4_ref, out_ref):
    e = e_ref[...]
    t = jnp.dot(e, a_ref[...], preferred_element_type=jnp.float32)
    fwfm = 0.5 * jnp.sum(e * t, axis=1, keepdims=True)
    h = jnp.maximum(
        jnp.dot(e, w1_ref[...], preferred_element_type=jnp.float32) + b1_ref[...], 0.0)
    h = jnp.maximum(
        jnp.dot(h, w2_ref[...], preferred_element_type=jnp.float32) + b2_ref[...], 0.0)
    h = jnp.maximum(
        jnp.dot(h, w3_ref[...], preferred_element_type=jnp.float32) + b3_ref[...], 0.0)
    o = jnp.dot(h, w4_ref[...], preferred_element_type=jnp.float32) + b4_ref[...]
    out_ref[...] = jax.nn.sigmoid(fwfm + o)


@jax.jit
def _tc_head(emb, a_big, W1, b1, W2, b2, W3, b3, W4, b4):
    return pl.pallas_call(
        _tc_body,
        out_shape=jax.ShapeDtypeStruct((_B, 1), jnp.float32),
    )(emb, a_big, W1, b1, W2, b2, W3, b3, W4, b4)


def kernel(x, tables, field_cov, W1, b1, W2, b2, W3, b3, W4, b4, bias):
    tab_t = tables.transpose(0, 2, 1).reshape(_IN, _V)
    x_flat = x.reshape(-1)
    emb = _gather(tab_t, x_flat)

    sym = (field_cov.T + field_cov) * 0.5 - jnp.eye(_F, dtype=jnp.float32)
    a_big = jnp.kron(sym, jnp.eye(_D, dtype=jnp.float32))
    out = _tc_head(
        emb, a_big,
        W1, b1.reshape(1, -1), W2, b2.reshape(1, -1),
        W3, b3.reshape(1, -1), W4, (b4 + bias).reshape(1, 1),
    )
    return out[:, 0]
